# Initial kernel scaffold; baseline (speedup 1.0000x reference)
import jax
import jax.numpy as jnp
from jax.experimental import pallas as pl


def kernel(x, edge_index, Wl1, bl1, Wr1, g1, b1, Wl2, bl2, Wr2, g2, b2, Wlin, blin):
    raise NotImplementedError("write your pallas kernel here")


# speedup vs baseline: 2.8495x; 2.8495x over previous
import jax
import jax.numpy as jnp
from jax import lax
from jax.experimental import pallas as pl
from jax.experimental.pallas import tpu as pltpu
from jax.experimental.pallas import tpu_sc as plsc

N = 10000
E = 320000
D = 128
CW = 16
DA = D + CW
EPS = 1e-5

NC = 2
NS = 16
NW = NC * NS
K = 128
NCHUNK = 80
EPW = NCHUNK * K
EPAD = NW * EPW
PAD = EPAD - E
ACC_ROWS = 10112
RPS = ACC_ROWS // NS

_mesh = plsc.VectorSubcoreMesh(
    core_axis_name="c", subcore_axis_name="s", num_cores=NC, num_subcores=NS
)


def _agg_body(tbl_hbm, srcw_hbm, dstw_hbm, zs_hbm, out_hbm,
              src_v, dst_v, rows_v, acc, gsem):
    cid = lax.axis_index("c")
    sid = lax.axis_index("s")
    wid = sid * NC + cid
    base = sid * RPS
    pltpu.sync_copy(zs_hbm, acc.at[pl.ds(base, RPS)])
    plsc.subcore_barrier()

    @pl.loop(0, NCHUNK)
    def _(j):
        pltpu.sync_copy(srcw_hbm.at[wid].at[j], src_v)
        pltpu.sync_copy(dstw_hbm.at[wid].at[j], dst_v)
        pltpu.async_copy(tbl_hbm.at[src_v], rows_v, gsem).wait()
        pltpu.sync_copy(rows_v, acc.at[dst_v], add=True)

    plsc.subcore_barrier()
    pltpu.sync_copy(acc.at[pl.ds(base, RPS)], out_hbm.at[cid].at[pl.ds(base, RPS)])


def _aggregate(table_aug, srcw, dstw, zs):
    k = pl.kernel(
        _agg_body,
        out_type=jax.ShapeDtypeStruct((NC, ACC_ROWS, DA), jnp.float32),
        mesh=_mesh,
        scratch_types=[
            pltpu.VMEM((K,), jnp.int32),
            pltpu.VMEM((K,), jnp.int32),
            pltpu.VMEM((K, DA), jnp.float32),
            pltpu.VMEM_SHARED((ACC_ROWS, DA), jnp.float32),
            pltpu.SemaphoreType.DMA,
        ],
        compiler_params=pltpu.CompilerParams(use_tc_tiling_on_sc=False),
    )
    return k(table_aug, srcw, dstw, zs)


def _lin_body(x_ref, w_ref, o_ref):
    o_ref[...] = lax.dot_general(
        x_ref[...], w_ref[...], (((1,), (1,)), ((), ())),
        preferred_element_type=jnp.float32)


def _linear(x, w):
    return pl.pallas_call(
        _lin_body,
        out_shape=jax.ShapeDtypeStruct((x.shape[0], w.shape[0]), jnp.float32),
    )(x, w)


def _sage_tail(s_ref, xr_ref, wl_ref, bl_ref, g_ref, b_ref):
    s = s_ref[...]
    aggr = s[0, :N, :D] + s[1, :N, :D]
    cnt = s[0, :N, D:D + 1] + s[1, :N, D:D + 1]
    mean = aggr / jnp.maximum(cnt, 1.0)
    pre = lax.dot_general(
        mean, wl_ref[...], (((1,), (1,)), ((), ())),
        preferred_element_type=jnp.float32)
    pre = pre + bl_ref[...] + xr_ref[...]
    mu = jnp.mean(pre, axis=0, keepdims=True)
    var = jnp.mean((pre - mu) ** 2, axis=0, keepdims=True)
    hn = (pre - mu) * lax.rsqrt(var + EPS) * g_ref[...] + b_ref[...]
    return jnp.maximum(hn, 0.0)


def _dense1_body(s_ref, xr_ref, wl_ref, bl_ref, g_ref, b_ref, o_ref):
    o_ref[:, :D] = _sage_tail(s_ref, xr_ref, wl_ref, bl_ref, g_ref, b_ref)
    o_ref[:, D:] = jnp.ones((N, CW), jnp.float32)


def _dense1(sums, xr, wl, bl, g, b):
    return pl.pallas_call(
        _dense1_body,
        out_shape=jax.ShapeDtypeStruct((N, DA), jnp.float32),
    )(sums, xr, wl.reshape(D, D), bl.reshape(1, D),
      g.reshape(1, D), b.reshape(1, D))


def _dense2_body(s_ref, xr_ref, wl_ref, bl_ref, g_ref, b_ref,
                 wlin_ref, blin_ref, o_ref):
    h = _sage_tail(s_ref, xr_ref, wl_ref, bl_ref, g_ref, b_ref)
    o_ref[...] = lax.dot_general(
        h, wlin_ref[...], (((1,), (1,)), ((), ())),
        preferred_element_type=jnp.float32) + blin_ref[...]


def _dense2(sums, xr, wl, bl, g, b, wlin, blin):
    return pl.pallas_call(
        _dense2_body,
        out_shape=jax.ShapeDtypeStruct((N, wlin.shape[0]), jnp.float32),
    )(sums, xr, wl.reshape(D, D), bl.reshape(1, D),
      g.reshape(1, D), b.reshape(1, D), wlin, blin.reshape(1, -1))


def kernel(x, edge_index, Wl1, bl1, Wr1, g1, b1, Wl2, bl2, Wr2, g2, b2, Wlin, blin):
    src = edge_index[0]
    dst = edge_index[1]
    srcw = jnp.concatenate([src, jnp.zeros((PAD,), jnp.int32)]).reshape(NW, NCHUNK, K)
    dstw = jnp.concatenate([dst, jnp.full((PAD,), N, jnp.int32)]).reshape(NW, NCHUNK, K)
    zs = jnp.zeros((RPS, DA), jnp.float32)
    x_aug = jnp.concatenate([x, jnp.ones((N, CW), jnp.float32)], axis=1)

    sums1 = _aggregate(x_aug, srcw, dstw, zs)
    xr1 = _linear(x, Wr1)
    h_aug = _dense1(sums1, xr1, Wl1, bl1, g1, b1)

    sums2 = _aggregate(h_aug, srcw, dstw, zs)
    hr2 = _linear(h_aug[:, :D], Wr2)
    return _dense2(sums2, hr2, Wl2, bl2, g2, b2, Wlin, blin)


# speedup vs baseline: 5.8743x; 2.0615x over previous
import jax
import jax.numpy as jnp
from jax import lax
from jax.experimental import pallas as pl
from jax.experimental.pallas import tpu as pltpu
from jax.experimental.pallas import tpu_sc as plsc

N = 10000
E = 320000
D = 128
CW = 16
DA = D + CW
EPS = 1e-5

NC = 2
NS = 16
NW = NC * NS
K = 96
IB = 21
NB = 5
NCHUNK = IB * NB
EPW = NCHUNK * K
EPAD = NW * EPW
PAD = EPAD - E
ACC_ROWS = 10112
RPS = ACC_ROWS // NS

_mesh = plsc.VectorSubcoreMesh(
    core_axis_name="c", subcore_axis_name="s", num_cores=NC, num_subcores=NS
)


def _agg_body(tbl_hbm, srcw_hbm, dstw_hbm, zs_hbm, out_hbm,
              srcb, dstb, rows0, rows1, acc, gsem0, gsem1, ssem0, ssem1):
    cid = lax.axis_index("c")
    sid = lax.axis_index("s")
    wid = sid * NC + cid
    base = sid * RPS
    pltpu.sync_copy(zs_hbm, acc.at[pl.ds(base, RPS)])
    plsc.subcore_barrier()

    rows = (rows0, rows1)
    gsems = (gsem0, gsem1)
    ssems = (ssem0, ssem1)

    @pl.loop(0, NB)
    def _(bk):
        pltpu.sync_copy(srcw_hbm.at[wid].at[pl.ds(bk * IB, IB)], srcb)
        pltpu.sync_copy(dstw_hbm.at[wid].at[pl.ds(bk * IB, IB)], dstb)
        d_g = {}
        d_s = {}
        d_g[0] = pltpu.async_copy(tbl_hbm.at[srcb.at[0]], rows[0], gsems[0])
        for i in range(IB):
            b = i & 1
            nb = b ^ 1
            if i + 1 < IB:
                if i >= 1:
                    d_s[i - 1].wait()
                d_g[i + 1] = pltpu.async_copy(
                    tbl_hbm.at[srcb.at[i + 1]], rows[nb], gsems[nb])
            d_g[i].wait()
            d_s[i] = pltpu.async_copy(rows[b], acc.at[dstb.at[i]],
                                      ssems[b], add=True)
        d_s[IB - 2].wait()
        d_s[IB - 1].wait()

    plsc.subcore_barrier()
    pltpu.sync_copy(acc.at[pl.ds(base, RPS)], out_hbm.at[cid].at[pl.ds(base, RPS)])


def _aggregate(table_aug, srcw, dstw, zs):
    k = pl.kernel(
        _agg_body,
        out_type=jax.ShapeDtypeStruct((NC, ACC_ROWS, DA), jnp.float32),
        mesh=_mesh,
        scratch_types=[
            pltpu.VMEM((IB, K), jnp.int32),
            pltpu.VMEM((IB, K), jnp.int32),
            pltpu.VMEM((K, DA), jnp.float32),
            pltpu.VMEM((K, DA), jnp.float32),
            pltpu.VMEM_SHARED((ACC_ROWS, DA), jnp.float32),
            pltpu.SemaphoreType.DMA,
            pltpu.SemaphoreType.DMA,
            pltpu.SemaphoreType.DMA,
            pltpu.SemaphoreType.DMA,
        ],
        compiler_params=pltpu.CompilerParams(use_tc_tiling_on_sc=False),
    )
    return k(table_aug, srcw, dstw, zs)


def _lin_body(x_ref, w_ref, o_ref):
    o_ref[...] = lax.dot_general(
        x_ref[...], w_ref[...], (((1,), (1,)), ((), ())),
        preferred_element_type=jnp.float32)


def _linear(x, w):
    return pl.pallas_call(
        _lin_body,
        out_shape=jax.ShapeDtypeStruct((x.shape[0], w.shape[0]), jnp.float32),
    )(x, w)


def _sage_tail(s_ref, xr_ref, wl_ref, bl_ref, g_ref, b_ref):
    s = s_ref[...]
    aggr = s[0, :N, :D] + s[1, :N, :D]
    cnt = s[0, :N, D:D + 1] + s[1, :N, D:D + 1]
    mean = aggr / jnp.maximum(cnt, 1.0)
    pre = lax.dot_general(
        mean, wl_ref[...], (((1,), (1,)), ((), ())),
        preferred_element_type=jnp.float32)
    pre = pre + bl_ref[...] + xr_ref[...]
    mu = jnp.mean(pre, axis=0, keepdims=True)
    var = jnp.mean((pre - mu) ** 2, axis=0, keepdims=True)
    hn = (pre - mu) * lax.rsqrt(var + EPS) * g_ref[...] + b_ref[...]
    return jnp.maximum(hn, 0.0)


def _dense1_body(s_ref, xr_ref, wl_ref, bl_ref, g_ref, b_ref, o_ref):
    o_ref[:, :D] = _sage_tail(s_ref, xr_ref, wl_ref, bl_ref, g_ref, b_ref)
    o_ref[:, D:] = jnp.ones((N, CW), jnp.float32)


def _dense1(sums, xr, wl, bl, g, b):
    return pl.pallas_call(
        _dense1_body,
        out_shape=jax.ShapeDtypeStruct((N, DA), jnp.float32),
    )(sums, xr, wl.reshape(D, D), bl.reshape(1, D),
      g.reshape(1, D), b.reshape(1, D))


def _dense2_body(s_ref, xr_ref, wl_ref, bl_ref, g_ref, b_ref,
                 wlin_ref, blin_ref, o_ref):
    h = _sage_tail(s_ref, xr_ref, wl_ref, bl_ref, g_ref, b_ref)
    o_ref[...] = lax.dot_general(
        h, wlin_ref[...], (((1,), (1,)), ((), ())),
        preferred_element_type=jnp.float32) + blin_ref[...]


def _dense2(sums, xr, wl, bl, g, b, wlin, blin):
    return pl.pallas_call(
        _dense2_body,
        out_shape=jax.ShapeDtypeStruct((N, wlin.shape[0]), jnp.float32),
    )(sums, xr, wl.reshape(D, D), bl.reshape(1, D),
      g.reshape(1, D), b.reshape(1, D), wlin, blin.reshape(1, -1))


def kernel(x, edge_index, Wl1, bl1, Wr1, g1, b1, Wl2, bl2, Wr2, g2, b2, Wlin, blin):
    src = edge_index[0]
    dst = edge_index[1]
    srcw = jnp.concatenate([src, jnp.zeros((PAD,), jnp.int32)]).reshape(NW, NCHUNK, K)
    dstw = jnp.concatenate([dst, jnp.full((PAD,), N, jnp.int32)]).reshape(NW, NCHUNK, K)
    zs = jnp.zeros((RPS, DA), jnp.float32)
    x_aug = jnp.concatenate([x, jnp.ones((N, CW), jnp.float32)], axis=1)

    sums1 = _aggregate(x_aug, srcw, dstw, zs)
    xr1 = _linear(x, Wr1)
    h_aug = _dense1(sums1, xr1, Wl1, bl1, g1, b1)

    sums2 = _aggregate(h_aug, srcw, dstw, zs)
    hr2 = _linear(h_aug[:, :D], Wr2)
    return _dense2(sums2, hr2, Wl2, bl2, g2, b2, Wlin, blin)
